# Initial kernel scaffold; baseline (speedup 1.0000x reference)
import jax
import jax.numpy as jnp
from jax.experimental import pallas as pl


def kernel(x, edge_index, batch, W0, b0, W1, b1, W2, b2, lin1_W, lin1_b, bn_gamma, bn_beta, lin2_W, lin2_b):
    raise NotImplementedError("write your pallas kernel here")


# speedup vs baseline: 9.8517x; 9.8517x over previous
import functools

import jax
import jax.numpy as jnp
from jax import lax
from jax.experimental import pallas as pl
from jax.experimental.pallas import tpu as pltpu
from jax.experimental.pallas import tpu_sc as plsc

N = 10000
E = 320000
D = 128
H = 128
G = 64

NC = 2
NS = 16
NW = NC * NS

NP = 10112
CH = 128
EPW = 10112
NCHUNK = EPW // CH
EP = EPW * NW
RPS = NP // NS

_mesh = plsc.VectorSubcoreMesh(core_axis_name="c", subcore_axis_name="s")


@functools.partial(
    pl.kernel,
    out_type=jax.ShapeDtypeStruct((NC, NP), jnp.float32),
    mesh=_mesh,
    scratch_types=[
        pltpu.VMEM((NCHUNK, CH), jnp.int32),
        pltpu.VMEM((CH,), jnp.float32),
        pltpu.VMEM_SHARED((NP,), jnp.float32),
        pltpu.SemaphoreType.DMA,
    ],
)
def _deg_sc(col_hbm, ones_hbm, zeros_hbm, out_hbm, colbuf, ones_v, deg_sh, sem):
    cid = lax.axis_index("c")
    sid = lax.axis_index("s")
    wid = sid * NC + cid

    @pl.when(sid == 0)
    def _():
        pltpu.sync_copy(zeros_hbm, deg_sh)

    pltpu.sync_copy(ones_hbm, ones_v)
    pltpu.sync_copy(col_hbm.at[wid], colbuf)
    plsc.subcore_barrier()

    def body(ci, carry):
        pltpu.async_copy(ones_v, deg_sh.at[colbuf.at[ci]], sem, add=True).wait()
        return carry

    lax.fori_loop(0, NCHUNK, body, 0)
    plsc.subcore_barrier()

    @pl.when(sid == 0)
    def _():
        pltpu.sync_copy(deg_sh, out_hbm.at[cid])


@functools.partial(
    pl.kernel,
    out_type=jax.ShapeDtypeStruct((NC, NP, H), jnp.float32),
    mesh=_mesh,
    scratch_types=[
        pltpu.VMEM((1, CH), jnp.int32),
        pltpu.VMEM((1, CH), jnp.int32),
        pltpu.VMEM((CH, H), jnp.float32),
        pltpu.VMEM_SHARED((NP, H), jnp.float32),
        pltpu.SemaphoreType.DMA,
        pltpu.SemaphoreType.DMA,
    ],
)
def _agg_sc(hs_hbm, row_hbm, col_hbm, zeros_hbm, out_hbm,
            rowbuf, colbuf, rows, agg_sh, gsem, ssem):
    cid = lax.axis_index("c")
    sid = lax.axis_index("s")
    wid = sid * NC + cid

    pltpu.sync_copy(zeros_hbm.at[pl.ds(sid * RPS, RPS)],
                    agg_sh.at[pl.ds(sid * RPS, RPS)])
    plsc.subcore_barrier()

    def body(ci, carry):
        pltpu.sync_copy(row_hbm.at[wid, ci], rowbuf.at[0])
        pltpu.sync_copy(col_hbm.at[wid, ci], colbuf.at[0])
        pltpu.async_copy(hs_hbm.at[rowbuf.at[0]], rows, gsem).wait()
        pltpu.async_copy(rows, agg_sh.at[colbuf.at[0]], ssem,
                         add=True).wait()
        return carry

    lax.fori_loop(0, NCHUNK, body, 0)
    plsc.subcore_barrier()

    pltpu.sync_copy(agg_sh.at[pl.ds(sid * RPS, RPS)],
                    out_hbm.at[cid, pl.ds(sid * RPS, RPS)])


def _prep_body(deg_ref, x_ref, w_ref, dinv_ref, hs_ref):
    deg = deg_ref[0] + deg_ref[1] + 1.0
    dinv = lax.rsqrt(deg)[:, None]
    dinv_ref[...] = dinv
    hs_ref[...] = jnp.dot(x_ref[...], w_ref[...],
                          preferred_element_type=jnp.float32) * dinv


_prep_tc = pl.pallas_call(
    _prep_body,
    out_shape=(jax.ShapeDtypeStruct((NP, 1), jnp.float32),
               jax.ShapeDtypeStruct((NP, H), jnp.float32)),
)


def _mid_body(agg_ref, hs_ref, dinv_ref, b_ref, w_ref, out_ref):
    dinv = dinv_ref[...]
    t = (agg_ref[0] + agg_ref[1] + hs_ref[...]) * dinv + b_ref[...]
    t = jnp.maximum(t, 0.0)
    out_ref[...] = jnp.dot(t, w_ref[...],
                           preferred_element_type=jnp.float32) * dinv


_mid_tc = pl.pallas_call(
    _mid_body,
    out_shape=jax.ShapeDtypeStruct((NP, H), jnp.float32),
)


def _final_body(agg_ref, hs_ref, dinv_ref, b_ref, batch_ref, l1w_ref, l1b_ref,
                gam_ref, beta_ref, l2w_ref, l2b_ref, out_ref):
    h3 = (agg_ref[0] + agg_ref[1] + hs_ref[...]) * dinv_ref[...] + b_ref[...]
    onehot = (lax.broadcasted_iota(jnp.int32, (G, NP), 0)
              == batch_ref[...]).astype(jnp.float32)
    gp = jnp.dot(onehot, h3, preferred_element_type=jnp.float32,
                 precision=lax.Precision.HIGHEST)
    z = jnp.dot(gp, l1w_ref[...], preferred_element_type=jnp.float32)
    z = jnp.maximum(z + l1b_ref[...], 0.0)
    mean = jnp.mean(z, axis=0, keepdims=True)
    var = jnp.mean((z - mean) ** 2, axis=0, keepdims=True)
    z = (z - mean) / jnp.sqrt(var + 1e-5) * gam_ref[...] + beta_ref[...]
    out_ref[...] = jnp.dot(z, l2w_ref[...],
                           preferred_element_type=jnp.float32) + l2b_ref[...]


_final_tc = pl.pallas_call(
    _final_body,
    out_shape=jax.ShapeDtypeStruct((G, 1), jnp.float32),
)


def kernel(x, edge_index, batch, W0, b0, W1, b1, W2, b2,
           lin1_W, lin1_b, bn_gamma, bn_beta, lin2_W, lin2_b):
    pad_e = EP - E
    rowp = jnp.concatenate(
        [edge_index[0], jnp.zeros((pad_e,), jnp.int32)]).reshape(NW, NCHUNK, CH)
    colp = jnp.concatenate(
        [edge_index[1], jnp.full((pad_e,), N, jnp.int32)]).reshape(NW, NCHUNK, CH)
    xp = jnp.pad(x, ((0, NP - N), (0, 0)))
    batchp = jnp.pad(batch, (0, NP - N), constant_values=G).reshape(1, NP)
    zeros2d = jnp.zeros((NP, H), jnp.float32)
    zeros1d = jnp.zeros((NP,), jnp.float32)
    ones_ch = jnp.ones((CH,), jnp.float32)

    deg2 = _deg_sc(colp, ones_ch, zeros1d)
    dinv, hs = _prep_tc(deg2, xp, W0)
    agg = _agg_sc(hs, rowp, colp, zeros2d)
    hs = _mid_tc(agg, hs, dinv, b0.reshape(1, H), W1)
    agg = _agg_sc(hs, rowp, colp, zeros2d)
    hs = _mid_tc(agg, hs, dinv, b1.reshape(1, H), W2)
    agg = _agg_sc(hs, rowp, colp, zeros2d)
    return _final_tc(agg, hs, dinv, b2.reshape(1, H), batchp,
                     lin1_W, lin1_b.reshape(1, 64),
                     bn_gamma.reshape(1, 64), bn_beta.reshape(1, 64),
                     lin2_W, lin2_b.reshape(1, 1))
